# baseline (speedup 1.0000x reference)
import jax
import jax.numpy as jnp
from jax import lax
from jax.experimental import pallas as pl
from jax.experimental.pallas import tpu as pltpu
from jax.experimental.pallas import tpu_sc as plsc

PERIODS = 10000
FREQS = 32
BATCH = 4096
SEQ = 200
D = 2 * FREQS

NC = 2
NS = 16
NW = NC * NS

B_TOTAL = BATCH * SEQ
B_PER_W = B_TOTAL // NW
CHUNK = 800
N_CHUNKS = B_PER_W // CHUNK


def _gather_body(table_hbm, idx_hbm, out_hbm, idx_v, rows_v,
                 isem0, isem1, gsem0, gsem1, osem0, osem1):
    wid = lax.axis_index("s") * NC + lax.axis_index("c")
    w_base = wid * B_PER_W
    isem = (isem0, isem1)
    gsem = (gsem0, gsem1)
    osem = (osem0, osem1)

    def idx_start(i, b):
        base = pl.multiple_of(w_base + i * CHUNK, 8)
        pltpu.async_copy(idx_hbm.at[pl.ds(base, CHUNK)], idx_v.at[b], isem[b])

    def idx_wait(i, b):
        base = pl.multiple_of(w_base + i * CHUNK, 8)
        pltpu.make_async_copy(
            idx_hbm.at[pl.ds(base, CHUNK)], idx_v.at[b], isem[b]).wait()

    def gather(b):
        pltpu.async_copy(table_hbm.at[idx_v.at[b]], rows_v.at[b],
                         gsem[b]).wait()

    def out_start(i, b):
        base = pl.multiple_of(w_base + i * CHUNK, 8)
        pltpu.async_copy(rows_v.at[b], out_hbm.at[pl.ds(base, CHUNK)], osem[b])

    def out_wait(i, b):
        base = pl.multiple_of(w_base + i * CHUNK, 8)
        pltpu.make_async_copy(
            rows_v.at[b], out_hbm.at[pl.ds(base, CHUNK)], osem[b]).wait()

    idx_start(0, 0)
    idx_start(1, 1)
    for b in range(2):
        idx_wait(b, b)
        gather(b)
        out_start(b, b)
        idx_start(b + 2, b)

    def step(j, carry):
        for b in range(2):
            i = 2 * j + b
            idx_wait(i, b)
            out_wait(i - 2, b)
            gather(b)
            out_start(i, b)
            idx_start(i + 2, b)
        return carry

    lax.fori_loop(1, N_CHUNKS // 2 - 1, step, 0)

    for b in range(2):
        i = N_CHUNKS - 2 + b
        idx_wait(i, b)
        out_wait(i - 2, b)
        gather(b)
        out_start(i, b)
    for b in range(2):
        out_wait(N_CHUNKS - 2 + b, b)


def _tile_body(in_ref, out_ref):
    eye = jnp.eye(128, dtype=jnp.float32)
    for m in range(4):
        mt = lax.dot_general(in_ref[pl.ds(m * 128, 128), :], eye,
                             (((0,), (0,)), ((), ())),
                             preferred_element_type=jnp.float32)
        out_ref[0, :, 2 * m, :, :] = mt[0:64].reshape(8, 8, 128)
        out_ref[0, :, 2 * m + 1, :, :] = mt[64:128].reshape(8, 8, 128)


@jax.jit
def _expand(tc, embedding):
    idx = (tc.T.reshape(SEQ, 16, 2, 128).transpose(0, 1, 3, 2)
           .reshape(-1).astype(jnp.int32))
    mesh = plsc.VectorSubcoreMesh(core_axis_name="c", subcore_axis_name="s")
    rows = pl.kernel(
        _gather_body,
        out_type=jax.ShapeDtypeStruct((B_TOTAL, D), jnp.float32),
        mesh=mesh,
        scratch_types=[
            pltpu.VMEM((2, CHUNK), jnp.int32),
            pltpu.VMEM((2, CHUNK, D), jnp.float32),
        ] + [pltpu.SemaphoreType.DMA] * 6,
        compiler_params=pltpu.CompilerParams(use_tc_tiling_on_sc=False),
    )(embedding, idx)

    rows2 = rows.reshape(B_TOTAL // 2, 2 * D)
    out5 = pl.pallas_call(
        _tile_body,
        grid=(SEQ, 4),
        in_specs=[pl.BlockSpec((512, 128), lambda i, j: (i * 4 + j, 0))],
        out_specs=pl.BlockSpec((1, 8, 8, 8, 128),
                               lambda i, j: (i, 0, j, 0, 0)),
        out_shape=jax.ShapeDtypeStruct((SEQ, 8, BATCH // 128, 8, 128),
                                       jnp.float32),
    )(rows2)
    return out5.transpose(2, 4, 0, 1, 3).reshape(BATCH, SEQ, D)


def kernel(tc, embedding):
    return _expand(tc, embedding)


# speedup vs baseline: 1.5246x; 1.5246x over previous
import jax
import jax.numpy as jnp
from jax import lax
from jax.experimental import pallas as pl
from jax.experimental.pallas import tpu as pltpu
from jax.experimental.pallas import tpu_sc as plsc

PERIODS = 10000
FREQS = 32
BATCH = 4096
SEQ = 200
D = 2 * FREQS

NC = 2
NS = 16
NW = NC * NS

B_TOTAL = BATCH * SEQ
B_PER_W = B_TOTAL // NW
CHUNK = 800
N_CHUNKS = B_PER_W // CHUNK


def _gather_body(table_hbm, idx_hbm, out_hbm, idx_v, rows_v,
                 isem0, isem1, gsem0, gsem1, osem0, osem1):
    wid = lax.axis_index("s") * NC + lax.axis_index("c")
    w_base = wid * B_PER_W
    isem = (isem0, isem1)
    gsem = (gsem0, gsem1)
    osem = (osem0, osem1)

    def idx_start(i, b):
        base = pl.multiple_of(w_base + i * CHUNK, 8)
        pltpu.async_copy(idx_hbm.at[pl.ds(base, CHUNK)], idx_v.at[b], isem[b])

    def idx_wait(i, b):
        base = pl.multiple_of(w_base + i * CHUNK, 8)
        pltpu.make_async_copy(
            idx_hbm.at[pl.ds(base, CHUNK)], idx_v.at[b], isem[b]).wait()

    def gather(b):
        pltpu.async_copy(table_hbm.at[idx_v.at[b]], rows_v.at[b],
                         gsem[b]).wait()

    def out_start(i, b):
        base = pl.multiple_of(w_base + i * CHUNK, 8)
        pltpu.async_copy(rows_v.at[b], out_hbm.at[pl.ds(base, CHUNK)], osem[b])

    def out_wait(i, b):
        base = pl.multiple_of(w_base + i * CHUNK, 8)
        pltpu.make_async_copy(
            rows_v.at[b], out_hbm.at[pl.ds(base, CHUNK)], osem[b]).wait()

    idx_start(0, 0)
    idx_start(1, 1)
    for b in range(2):
        idx_wait(b, b)
        gather(b)
        out_start(b, b)
        idx_start(b + 2, b)

    def step(j, carry):
        for b in range(2):
            i = 2 * j + b
            idx_wait(i, b)
            out_wait(i - 2, b)
            gather(b)
            out_start(i, b)
            idx_start(i + 2, b)
        return carry

    lax.fori_loop(1, N_CHUNKS // 2 - 1, step, 0)

    for b in range(2):
        i = N_CHUNKS - 2 + b
        idx_wait(i, b)
        out_wait(i - 2, b)
        gather(b)
        out_start(i, b)
    for b in range(2):
        out_wait(N_CHUNKS - 2 + b, b)


def _tile_body(in_ref, out_ref):
    eye = jnp.eye(128, dtype=jnp.float32)
    for m in range(4):
        mt = lax.dot_general(in_ref[pl.ds(m * 128, 128), :], eye,
                             (((0,), (0,)), ((), ())),
                             preferred_element_type=jnp.float32)
        out_ref[0, :, 2 * m, :, :] = mt[0:64].reshape(8, 8, 128)
        out_ref[0, :, 2 * m + 1, :, :] = mt[64:128].reshape(8, 8, 128)


@jax.jit
def _expand(tc, embedding):
    idx = tc.T.reshape(-1).astype(jnp.int32)
    mesh = plsc.VectorSubcoreMesh(core_axis_name="c", subcore_axis_name="s")
    rows = pl.kernel(
        _gather_body,
        out_type=jax.ShapeDtypeStruct((B_TOTAL, D), jnp.float32),
        mesh=mesh,
        scratch_types=[
            pltpu.VMEM((2, CHUNK), jnp.int32),
            pltpu.VMEM((2, CHUNK, D), jnp.float32),
        ] + [pltpu.SemaphoreType.DMA] * 6,
        compiler_params=pltpu.CompilerParams(use_tc_tiling_on_sc=False),
    )(embedding, idx)

    return rows.reshape(SEQ, BATCH, D).transpose(1, 0, 2)


def kernel(tc, embedding):
    return _expand(tc, embedding)
